# baseline (speedup 1.0000x reference)
import functools

import jax
import jax.numpy as jnp
from jax.experimental import pallas as pl
from jax.experimental.pallas import tpu as pltpu

_B, _T, _D, _R = 4, 2048, 512, 1024
_N_INNER, _N_B = 9, 2
_RATIO = 1.0 / 5.0
_NB = _N_B + _N_INNER + _N_B
_RBLK = 128
_NRB = _R // _RBLK
_MI_U = 16

_SELU_ALPHA = 1.6732632423543772
_SELU_SCALE = 1.0507009873554805


def _selu(x):
    return _SELU_SCALE * jnp.where(x > 0, x, _SELU_ALPHA * (jnp.exp(x) - 1.0))


def _roi_kernel(pos_ref,
                feats_ref,
                wl_ref,
                wi_ref,
                wr_ref,
                bl_ref, bi_ref, br_ref,
                out_ref,
                tile_ref):
    b = pl.program_id(0)
    rb = pl.program_id(1)
    blk_base = ((b * _NRB + rb) * _NB) * _RBLK

    def gather_group(g, carry):
        for u in range(_MI_U):
            mi = g * _MI_U + u
            for nb in range(_NB):
                idx = blk_base + nb * _RBLK + mi
                p = pos_ref[idx]
                lo = jnp.minimum(p.astype(jnp.int32), _T - 2)
                w = p - lo.astype(jnp.float32)
                pair = feats_ref[pl.ds(lo, 2), 0, :]
                v = pair[0:1, :] + w * (pair[1:2, :] - pair[0:1, :])
                tile_ref[pl.ds(nb * _RBLK + mi, 1), 0, :] = v
        return carry

    jax.lax.fori_loop(0, _RBLK // _MI_U, gather_group, 0)

    def xv(nb):
        return tile_ref[pl.ds(nb * _RBLK, _RBLK), 0, :]

    def dot(x, w):
        return jnp.dot(x, w, preferred_element_type=jnp.float32)

    acc_l = dot(xv(0), wl_ref[0])
    for j in range(1, 7):
        acc_l = acc_l + dot(xv(j), wl_ref[j])
    left = _selu(acc_l + bl_ref[0, :])

    acc_r = dot(xv(6), wl_ref[0])
    for j in range(1, 7):
        acc_r = acc_r + dot(xv(6 + j), wl_ref[j])
    right = _selu(acc_r + bl_ref[0, :])

    part1 = dot(right - left, wr_ref[0])

    acc_i = dot(xv(2), wi_ref[0])
    for j in range(1, 9):
        acc_i = acc_i + dot(xv(2 + j), wi_ref[j])
    inner = _selu(acc_i + bi_ref[0, :])

    out_ref[0, :, :] = _selu(part1 + dot(inner, wr_ref[1]) + br_ref[0, :])


def _bin_centers(a, b, n):
    i = (jnp.arange(n, dtype=a.dtype) + 0.5) / n
    return a[..., None] + i * (b - a)[..., None]


@jax.jit
def kernel(features, start_rois, end_rois, rois, rois_mask, rois_pos_emb,
           W_left, b_left, W_inner, b_inner, W_roi, b_roi):
    del start_rois, end_rois, rois_mask, rois_pos_emb

    s, e = rois[..., 0], rois[..., 1]
    ext = _RATIO * (e - s)
    pos = jnp.concatenate([
        _bin_centers(s - ext, s + ext, _N_B),
        _bin_centers(s, e, _N_INNER),
        _bin_centers(e - ext, e + ext, _N_B)], axis=-1)
    pos = jnp.clip(pos, 0.0, _T - 1)
    pos_flat = pos.reshape(_B, _NRB, _RBLK, _NB).transpose(0, 1, 3, 2).reshape(-1)

    feats_p = features.reshape(_B * _T, 1, _D)
    wl = W_left.T.reshape(_D, 7, _D).transpose(1, 0, 2)
    wi = W_inner.T.reshape(_D, 9, _D).transpose(1, 0, 2)
    wr = W_roi.T.reshape(2, _D, _D)
    bl = b_left.reshape(1, _D)
    bi = b_inner.reshape(1, _D)
    br = b_roi.reshape(1, _D)

    out = pl.pallas_call(
        _roi_kernel,
        out_shape=jax.ShapeDtypeStruct((_B, _R, _D), jnp.float32),
        grid=(_B, _NRB),
        in_specs=[
            pl.BlockSpec(memory_space=pltpu.SMEM),
            pl.BlockSpec((_T, 1, _D), lambda b, rb: (b, 0, 0)),
            pl.BlockSpec((7, _D, _D), lambda b, rb: (0, 0, 0)),
            pl.BlockSpec((9, _D, _D), lambda b, rb: (0, 0, 0)),
            pl.BlockSpec((2, _D, _D), lambda b, rb: (0, 0, 0)),
            pl.BlockSpec((1, _D), lambda b, rb: (0, 0)),
            pl.BlockSpec((1, _D), lambda b, rb: (0, 0)),
            pl.BlockSpec((1, _D), lambda b, rb: (0, 0)),
        ],
        out_specs=pl.BlockSpec((1, _RBLK, _D), lambda b, rb: (b, rb, 0)),
        scratch_shapes=[pltpu.VMEM((_NB * _RBLK, 1, _D), jnp.float32)],
        compiler_params=pltpu.CompilerParams(
            dimension_semantics=("parallel", "arbitrary"),
            vmem_limit_bytes=60 * 1024 * 1024,
        ),
        name="roi_relation_fused",
    )(pos_flat, feats_p, wl, wi, wr, bl, bi, br)
    return out


# speedup vs baseline: 1.0004x; 1.0004x over previous
import functools

import jax
import jax.numpy as jnp
from jax.experimental import pallas as pl
from jax.experimental.pallas import tpu as pltpu

_B, _T, _D, _R = 4, 2048, 512, 1024
_N_INNER, _N_B = 9, 2
_RATIO = 1.0 / 5.0
_NB = _N_B + _N_INNER + _N_B
_RBLK = 128
_NRB = _R // _RBLK
_MI_U = 16

_SELU_ALPHA = 1.6732632423543772
_SELU_SCALE = 1.0507009873554805


def _selu(x):
    return _SELU_SCALE * jnp.where(x > 0, x, _SELU_ALPHA * (jnp.exp(x) - 1.0))


def _roi_kernel(pos_ref,
                feats_ref,
                wl_ref,
                wi_ref,
                wr_ref,
                bl_ref, bi_ref, br_ref,
                out_ref,
                tile_ref):
    b = pl.program_id(0)
    rb = pl.program_id(1)
    blk_base = ((b * _NRB + rb) * _RBLK) * _NB

    def gather_group(g, carry):
        for u in range(_MI_U):
            mi = g * _MI_U + u
            for nb in range(_NB):
                idx = blk_base + mi * _NB + nb
                p = pos_ref[idx]
                lo = jnp.minimum(p.astype(jnp.int32), _T - 2)
                w = p - lo.astype(jnp.float32)
                pair = feats_ref[pl.ds(lo, 2), 0, :]
                v = pair[0:1, :] + w * (pair[1:2, :] - pair[0:1, :])
                tile_ref[pl.ds(nb * _RBLK + mi, 1), 0, :] = v
        return carry

    jax.lax.fori_loop(0, _RBLK // _MI_U, gather_group, 0)

    def xv(nb):
        return tile_ref[pl.ds(nb * _RBLK, _RBLK), 0, :]

    def dot(x, w):
        return jnp.dot(x, w, preferred_element_type=jnp.float32)

    acc_l = dot(xv(0), wl_ref[0])
    for j in range(1, 7):
        acc_l = acc_l + dot(xv(j), wl_ref[j])
    left = _selu(acc_l + bl_ref[0, :])

    acc_r = dot(xv(6), wl_ref[0])
    for j in range(1, 7):
        acc_r = acc_r + dot(xv(6 + j), wl_ref[j])
    right = _selu(acc_r + bl_ref[0, :])

    part1 = dot(right - left, wr_ref[0])

    acc_i = dot(xv(2), wi_ref[0])
    for j in range(1, 9):
        acc_i = acc_i + dot(xv(2 + j), wi_ref[j])
    inner = _selu(acc_i + bi_ref[0, :])

    out_ref[0, :, :] = _selu(part1 + dot(inner, wr_ref[1]) + br_ref[0, :])


def _bin_centers(a, b, n):
    i = (jnp.arange(n, dtype=a.dtype) + 0.5) / n
    return a[..., None] + i * (b - a)[..., None]


@jax.jit
def kernel(features, start_rois, end_rois, rois, rois_mask, rois_pos_emb,
           W_left, b_left, W_inner, b_inner, W_roi, b_roi):
    del start_rois, end_rois, rois_mask, rois_pos_emb

    s, e = rois[..., 0], rois[..., 1]
    ext = _RATIO * (e - s)
    pos = jnp.concatenate([
        _bin_centers(s - ext, s + ext, _N_B),
        _bin_centers(s, e, _N_INNER),
        _bin_centers(e - ext, e + ext, _N_B)], axis=-1)
    pos = jnp.clip(pos, 0.0, _T - 1)
    pos_flat = pos.reshape(-1)

    feats_p = features.reshape(_B * _T, 1, _D)
    wl = W_left.reshape(_D, _D, 7).transpose(2, 1, 0)
    wi = W_inner.reshape(_D, _D, 9).transpose(2, 1, 0)
    wr = W_roi.T.reshape(2, _D, _D)
    bl = b_left.reshape(1, _D)
    bi = b_inner.reshape(1, _D)
    br = b_roi.reshape(1, _D)

    out = pl.pallas_call(
        _roi_kernel,
        out_shape=jax.ShapeDtypeStruct((_B, _R, _D), jnp.float32),
        grid=(_B, _NRB),
        in_specs=[
            pl.BlockSpec(memory_space=pltpu.SMEM),
            pl.BlockSpec((_T, 1, _D), lambda b, rb: (b, 0, 0)),
            pl.BlockSpec((7, _D, _D), lambda b, rb: (0, 0, 0)),
            pl.BlockSpec((9, _D, _D), lambda b, rb: (0, 0, 0)),
            pl.BlockSpec((2, _D, _D), lambda b, rb: (0, 0, 0)),
            pl.BlockSpec((1, _D), lambda b, rb: (0, 0)),
            pl.BlockSpec((1, _D), lambda b, rb: (0, 0)),
            pl.BlockSpec((1, _D), lambda b, rb: (0, 0)),
        ],
        out_specs=pl.BlockSpec((1, _RBLK, _D), lambda b, rb: (b, rb, 0)),
        scratch_shapes=[pltpu.VMEM((_NB * _RBLK, 1, _D), jnp.float32)],
        compiler_params=pltpu.CompilerParams(
            dimension_semantics=("parallel", "arbitrary"),
            vmem_limit_bytes=60 * 1024 * 1024,
        ),
        name="roi_relation_fused",
    )(pos_flat, feats_p, wl, wi, wr, bl, bi, br)
    return out


# speedup vs baseline: 1.3910x; 1.3905x over previous
import functools

import jax
import jax.numpy as jnp
from jax.experimental import pallas as pl
from jax.experimental.pallas import tpu as pltpu

_B, _T, _D, _R = 4, 2048, 512, 1024
_N_INNER, _N_B = 9, 2
_RATIO = 1.0 / 5.0
_NB = _N_B + _N_INNER + _N_B
_RBLK = 128
_NRB = _R // _RBLK
_MI_U = 16

_SELU_ALPHA = 1.6732632423543772
_SELU_SCALE = 1.0507009873554805


def _selu(x):
    return _SELU_SCALE * jnp.where(x > 0, x, _SELU_ALPHA * (jnp.exp(x) - 1.0))


def _roi_kernel(lo_ref,
                w_ref,
                feats_ref,
                wl_ref,
                wi_ref,
                wr_ref,
                bl_ref, bi_ref, br_ref,
                out_ref,
                tile_ref):
    b = pl.program_id(0)
    rb = pl.program_id(1)
    blk_base = ((b * _NRB + rb) * _RBLK) * _NB

    def gather_group(g, carry):
        for u in range(_MI_U):
            mi = g * _MI_U + u
            for nb in range(_NB):
                idx = blk_base + mi * _NB + nb
                lo = lo_ref[idx]
                w = w_ref[idx]
                pair = feats_ref[pl.ds(lo, 2), 0, :]
                v = pair[0:1, :] + w * (pair[1:2, :] - pair[0:1, :])
                tile_ref[pl.ds(nb * _RBLK + mi, 1), 0, :] = v
        return carry

    jax.lax.fori_loop(0, _RBLK // _MI_U, gather_group, 0)

    def xv(nb):
        return tile_ref[pl.ds(nb * _RBLK, _RBLK), 0, :]

    def dot(x, w):
        return jnp.dot(x, w, preferred_element_type=jnp.float32)

    acc_l = dot(xv(0), wl_ref[0])
    for j in range(1, 7):
        acc_l = acc_l + dot(xv(j), wl_ref[j])
    left = _selu(acc_l + bl_ref[0, :])

    acc_r = dot(xv(6), wl_ref[0])
    for j in range(1, 7):
        acc_r = acc_r + dot(xv(6 + j), wl_ref[j])
    right = _selu(acc_r + bl_ref[0, :])

    part1 = dot(right - left, wr_ref[0])

    acc_i = dot(xv(2), wi_ref[0])
    for j in range(1, 9):
        acc_i = acc_i + dot(xv(2 + j), wi_ref[j])
    inner = _selu(acc_i + bi_ref[0, :])

    out_ref[0, :, :] = _selu(part1 + dot(inner, wr_ref[1]) + br_ref[0, :])


def _bin_centers(a, b, n):
    i = (jnp.arange(n, dtype=a.dtype) + 0.5) / n
    return a[..., None] + i * (b - a)[..., None]


@jax.jit
def kernel(features, start_rois, end_rois, rois, rois_mask, rois_pos_emb,
           W_left, b_left, W_inner, b_inner, W_roi, b_roi):
    del start_rois, end_rois, rois_mask, rois_pos_emb

    s, e = rois[..., 0], rois[..., 1]
    ext = _RATIO * (e - s)
    pos = jnp.concatenate([
        _bin_centers(s - ext, s + ext, _N_B),
        _bin_centers(s, e, _N_INNER),
        _bin_centers(e - ext, e + ext, _N_B)], axis=-1)
    pos = jnp.clip(pos, 0.0, _T - 1)
    lo = jnp.minimum(pos.astype(jnp.int32), _T - 2)
    w = pos - lo.astype(jnp.float32)
    lo_flat = lo.reshape(-1)
    w_flat = w.reshape(-1)

    feats_p = features.reshape(_B * _T, 1, _D)
    wl = W_left.reshape(_D, _D, 7).transpose(2, 1, 0)
    wi = W_inner.reshape(_D, _D, 9).transpose(2, 1, 0)
    wr = W_roi.T.reshape(2, _D, _D)
    bl = b_left.reshape(1, _D)
    bi = b_inner.reshape(1, _D)
    br = b_roi.reshape(1, _D)

    out = pl.pallas_call(
        _roi_kernel,
        out_shape=jax.ShapeDtypeStruct((_B, _R, _D), jnp.float32),
        grid=(_B, _NRB),
        in_specs=[
            pl.BlockSpec(memory_space=pltpu.SMEM),
            pl.BlockSpec(memory_space=pltpu.SMEM),
            pl.BlockSpec((_T, 1, _D), lambda b, rb: (b, 0, 0)),
            pl.BlockSpec((7, _D, _D), lambda b, rb: (0, 0, 0)),
            pl.BlockSpec((9, _D, _D), lambda b, rb: (0, 0, 0)),
            pl.BlockSpec((2, _D, _D), lambda b, rb: (0, 0, 0)),
            pl.BlockSpec((1, _D), lambda b, rb: (0, 0)),
            pl.BlockSpec((1, _D), lambda b, rb: (0, 0)),
            pl.BlockSpec((1, _D), lambda b, rb: (0, 0)),
        ],
        out_specs=pl.BlockSpec((1, _RBLK, _D), lambda b, rb: (b, rb, 0)),
        scratch_shapes=[pltpu.VMEM((_NB * _RBLK, 1, _D), jnp.float32)],
        compiler_params=pltpu.CompilerParams(
            dimension_semantics=("parallel", "arbitrary"),
            vmem_limit_bytes=60 * 1024 * 1024,
        ),
        name="roi_relation_fused",
    )(lo_flat, w_flat, feats_p, wl, wi, wr, bl, bi, br)
    return out


# speedup vs baseline: 1.4362x; 1.0325x over previous
import functools

import jax
import jax.numpy as jnp
from jax.experimental import pallas as pl
from jax.experimental.pallas import tpu as pltpu

_B, _T, _D, _R = 4, 2048, 512, 1024
_N_INNER, _N_B = 9, 2
_RATIO = 1.0 / 5.0
_NB = _N_B + _N_INNER + _N_B
_RBLK = 256
_NRB = _R // _RBLK
_MI_U = 16

_SELU_ALPHA = 1.6732632423543772
_SELU_SCALE = 1.0507009873554805


def _selu(x):
    return _SELU_SCALE * jnp.where(x > 0, x, _SELU_ALPHA * (jnp.exp(x) - 1.0))


def _roi_kernel(lo_ref,
                w_ref,
                feats_ref,
                wl_ref,
                wi_ref,
                wr_ref,
                bl_ref, bi_ref, br_ref,
                out_ref,
                tile_ref):
    b = pl.program_id(0)
    rb = pl.program_id(1)
    blk_base = ((b * _NRB + rb) * _RBLK) * _NB

    def gather_group(g, carry):
        for u in range(_MI_U):
            mi = g * _MI_U + u
            for nb in range(_NB):
                idx = blk_base + mi * _NB + nb
                lo = lo_ref[idx]
                w = w_ref[idx]
                pair = feats_ref[pl.ds(lo, 2), 0, :]
                v = pair[0:1, :] + w * (pair[1:2, :] - pair[0:1, :])
                tile_ref[pl.ds(nb * _RBLK + mi, 1), 0, :] = v
        return carry

    jax.lax.fori_loop(0, _RBLK // _MI_U, gather_group, 0)

    def xv(nb):
        return tile_ref[pl.ds(nb * _RBLK, _RBLK), 0, :]

    def dot(x, w):
        return jnp.dot(x, w, preferred_element_type=jnp.float32)

    acc_l = dot(xv(0), wl_ref[0])
    for j in range(1, 7):
        acc_l = acc_l + dot(xv(j), wl_ref[j])
    left = _selu(acc_l + bl_ref[0, :])

    acc_r = dot(xv(6), wl_ref[0])
    for j in range(1, 7):
        acc_r = acc_r + dot(xv(6 + j), wl_ref[j])
    right = _selu(acc_r + bl_ref[0, :])

    part1 = dot(right - left, wr_ref[0])

    acc_i = dot(xv(2), wi_ref[0])
    for j in range(1, 9):
        acc_i = acc_i + dot(xv(2 + j), wi_ref[j])
    inner = _selu(acc_i + bi_ref[0, :])

    out_ref[0, :, :] = _selu(part1 + dot(inner, wr_ref[1]) + br_ref[0, :])


def _bin_centers(a, b, n):
    i = (jnp.arange(n, dtype=a.dtype) + 0.5) / n
    return a[..., None] + i * (b - a)[..., None]


@jax.jit
def kernel(features, start_rois, end_rois, rois, rois_mask, rois_pos_emb,
           W_left, b_left, W_inner, b_inner, W_roi, b_roi):
    del start_rois, end_rois, rois_mask, rois_pos_emb

    s, e = rois[..., 0], rois[..., 1]
    ext = _RATIO * (e - s)
    pos = jnp.concatenate([
        _bin_centers(s - ext, s + ext, _N_B),
        _bin_centers(s, e, _N_INNER),
        _bin_centers(e - ext, e + ext, _N_B)], axis=-1)
    pos = jnp.clip(pos, 0.0, _T - 1)
    lo = jnp.minimum(pos.astype(jnp.int32), _T - 2)
    w = pos - lo.astype(jnp.float32)
    lo_flat = lo.reshape(-1)
    w_flat = w.reshape(-1)

    feats_p = features.reshape(_B * _T, 1, _D)
    wl = W_left.reshape(_D, _D, 7).transpose(2, 1, 0)
    wi = W_inner.reshape(_D, _D, 9).transpose(2, 1, 0)
    wr = W_roi.T.reshape(2, _D, _D)
    bl = b_left.reshape(1, _D)
    bi = b_inner.reshape(1, _D)
    br = b_roi.reshape(1, _D)

    out = pl.pallas_call(
        _roi_kernel,
        out_shape=jax.ShapeDtypeStruct((_B, _R, _D), jnp.float32),
        grid=(_B, _NRB),
        in_specs=[
            pl.BlockSpec(memory_space=pltpu.SMEM),
            pl.BlockSpec(memory_space=pltpu.SMEM),
            pl.BlockSpec((_T, 1, _D), lambda b, rb: (b, 0, 0)),
            pl.BlockSpec((7, _D, _D), lambda b, rb: (0, 0, 0)),
            pl.BlockSpec((9, _D, _D), lambda b, rb: (0, 0, 0)),
            pl.BlockSpec((2, _D, _D), lambda b, rb: (0, 0, 0)),
            pl.BlockSpec((1, _D), lambda b, rb: (0, 0)),
            pl.BlockSpec((1, _D), lambda b, rb: (0, 0)),
            pl.BlockSpec((1, _D), lambda b, rb: (0, 0)),
        ],
        out_specs=pl.BlockSpec((1, _RBLK, _D), lambda b, rb: (b, rb, 0)),
        scratch_shapes=[pltpu.VMEM((_NB * _RBLK, 1, _D), jnp.float32)],
        compiler_params=pltpu.CompilerParams(
            dimension_semantics=("parallel", "arbitrary"),
            vmem_limit_bytes=60 * 1024 * 1024,
        ),
        name="roi_relation_fused",
    )(lo_flat, w_flat, feats_p, wl, wi, wr, bl, bi, br)
    return out


# speedup vs baseline: 1.7312x; 1.2054x over previous
import functools

import jax
import jax.numpy as jnp
from jax.experimental import pallas as pl
from jax.experimental.pallas import tpu as pltpu

_B, _T, _D, _R = 4, 2048, 512, 1024
_N_INNER, _N_B = 9, 2
_RATIO = 1.0 / 5.0
_NB = _N_B + _N_INNER + _N_B
_RBLK = 256
_NRB = _R // _RBLK
_MI_U = 16

_SELU_ALPHA = 1.6732632423543772
_SELU_SCALE = 1.0507009873554805


def _selu(x):
    return _SELU_SCALE * jnp.where(x > 0, x, _SELU_ALPHA * (jnp.exp(x) - 1.0))


def _roi_kernel(lo_ref,
                w_ref,
                feats_ref,
                wl_ref,
                wi_ref,
                wr_ref,
                bl_ref, bi_ref, br_ref,
                out_ref,
                tile_ref,
                f1_ref):
    b = pl.program_id(0)
    rb = pl.program_id(1)
    blk_base = ((b * _NRB + rb) * _RBLK) * _NB

    @pl.when(rb == 0)
    def _():
        for c in range(_T // 256):
            f1_ref[pl.ds(c * 256, 256), 0, :] = feats_ref[0, pl.ds(c * 256, 256), :]

    def gather_group(g, carry):
        for u in range(_MI_U):
            mi = g * _MI_U + u
            for nb in range(_NB):
                idx = blk_base + mi * _NB + nb
                lo = lo_ref[idx]
                w = w_ref[idx]
                pair = f1_ref[pl.ds(lo, 2), 0, :]
                v = pair[0:1, :] + w * (pair[1:2, :] - pair[0:1, :])
                tile_ref[pl.ds(nb * _RBLK + mi, 1), 0, :] = v
        return carry

    jax.lax.fori_loop(0, _RBLK // _MI_U, gather_group, 0)

    def xv(nb):
        return tile_ref[pl.ds(nb * _RBLK, _RBLK), 0, :]

    def dot(x, w):
        return jnp.dot(x, w, preferred_element_type=jnp.float32)

    acc_l = dot(xv(0), wl_ref[0])
    for j in range(1, 7):
        acc_l = acc_l + dot(xv(j), wl_ref[j])
    left = _selu(acc_l + bl_ref[0, :])

    acc_r = dot(xv(6), wl_ref[0])
    for j in range(1, 7):
        acc_r = acc_r + dot(xv(6 + j), wl_ref[j])
    right = _selu(acc_r + bl_ref[0, :])

    part1 = dot(right - left, wr_ref[0])

    acc_i = dot(xv(2), wi_ref[0])
    for j in range(1, 9):
        acc_i = acc_i + dot(xv(2 + j), wi_ref[j])
    inner = _selu(acc_i + bi_ref[0, :])

    out_ref[0, :, :] = _selu(part1 + dot(inner, wr_ref[1]) + br_ref[0, :])


def _bin_centers(a, b, n):
    i = (jnp.arange(n, dtype=a.dtype) + 0.5) / n
    return a[..., None] + i * (b - a)[..., None]


@jax.jit
def kernel(features, start_rois, end_rois, rois, rois_mask, rois_pos_emb,
           W_left, b_left, W_inner, b_inner, W_roi, b_roi):
    del start_rois, end_rois, rois_mask, rois_pos_emb

    s, e = rois[..., 0], rois[..., 1]
    ext = _RATIO * (e - s)
    pos = jnp.concatenate([
        _bin_centers(s - ext, s + ext, _N_B),
        _bin_centers(s, e, _N_INNER),
        _bin_centers(e - ext, e + ext, _N_B)], axis=-1)
    pos = jnp.clip(pos, 0.0, _T - 1)
    lo = jnp.minimum(pos.astype(jnp.int32), _T - 2)
    w = pos - lo.astype(jnp.float32)
    lo_flat = lo.reshape(-1)
    w_flat = w.reshape(-1)

    wl = W_left.reshape(_D, _D, 7).transpose(2, 1, 0)
    wi = W_inner.reshape(_D, _D, 9).transpose(2, 1, 0)
    wr = W_roi.T.reshape(2, _D, _D)
    bl = b_left.reshape(1, _D)
    bi = b_inner.reshape(1, _D)
    br = b_roi.reshape(1, _D)

    out = pl.pallas_call(
        _roi_kernel,
        out_shape=jax.ShapeDtypeStruct((_B, _R, _D), jnp.float32),
        grid=(_B, _NRB),
        in_specs=[
            pl.BlockSpec(memory_space=pltpu.SMEM),
            pl.BlockSpec(memory_space=pltpu.SMEM),
            pl.BlockSpec((1, _T, _D), lambda b, rb: (b, 0, 0)),
            pl.BlockSpec((7, _D, _D), lambda b, rb: (0, 0, 0)),
            pl.BlockSpec((9, _D, _D), lambda b, rb: (0, 0, 0)),
            pl.BlockSpec((2, _D, _D), lambda b, rb: (0, 0, 0)),
            pl.BlockSpec((1, _D), lambda b, rb: (0, 0)),
            pl.BlockSpec((1, _D), lambda b, rb: (0, 0)),
            pl.BlockSpec((1, _D), lambda b, rb: (0, 0)),
        ],
        out_specs=pl.BlockSpec((1, _RBLK, _D), lambda b, rb: (b, rb, 0)),
        scratch_shapes=[pltpu.VMEM((_NB * _RBLK, 1, _D), jnp.float32),
                        pltpu.VMEM((_T, 1, _D), jnp.float32)],
        compiler_params=pltpu.CompilerParams(
            dimension_semantics=("parallel", "arbitrary"),
            vmem_limit_bytes=60 * 1024 * 1024,
        ),
        name="roi_relation_fused",
    )(lo_flat, w_flat, features, wl, wi, wr, bl, bi, br)
    return out
